# baseline (speedup 1.0000x reference)
import jax
import jax.numpy as jnp
from jax import lax
from jax.experimental import pallas as pl
from jax.experimental.pallas import tpu as pltpu
from jax.experimental.pallas import tpu_sc as plsc

BATCH = 16384
EMBED = 64
PAIRW = 128
HID1 = 128
HID2 = 64
CHUNK = 128

_info = plsc.get_sparse_core_info()
_NC, _NS = _info.num_cores, _info.num_subcores
_NW = _NC * _NS
_BPW = BATCH // _NW
_NCHUNK = _BPW // CHUNK

_TCOLS = 16384


def _tpose_body(tt, out):
    t = lax.dot_general(tt[...], jnp.eye(EMBED, dtype=jnp.float32),
                        (((0,), (0,)), ((), ())),
                        preferred_element_type=jnp.float32)
    t3 = t.reshape(_TCOLS // 2, 2, EMBED)
    out[...] = jnp.concatenate([t3[:, 0, :], t3[:, 1, :]], axis=1)


def _make_tpose(nrows):
    return pl.pallas_call(
        _tpose_body,
        grid=(pl.cdiv(nrows, _TCOLS),),
        in_specs=[pl.BlockSpec((EMBED, _TCOLS), lambda i: (0, i))],
        out_specs=pl.BlockSpec((_TCOLS // 2, PAIRW), lambda i: (i, 0)),
        out_shape=jax.ShapeDtypeStruct((nrows // 2, PAIRW), jnp.float32),
        compiler_params=pltpu.CompilerParams(
            dimension_semantics=("arbitrary",)),
    )


def _gather_body(uidx_hbm, iidx_hbm, utab_hbm, itab_hbm, up_hbm, ip_hbm,
                 idx_v, rows_v, sem):
    wid = lax.axis_index("s") * _NC + lax.axis_index("c")
    base = wid * _BPW
    row0 = wid * _NCHUNK

    def one_table(idx_hbm, tab_hbm, out_hbm):
        pltpu.sync_copy(idx_hbm.at[pl.ds(row0, _NCHUNK)], idx_v)
        copies = [
            pltpu.async_copy(tab_hbm.at[idx_v.at[j]],
                             rows_v.at[pl.ds(j * CHUNK, CHUNK)], sem)
            for j in range(_NCHUNK)
        ]
        for c in copies:
            c.wait()
        pltpu.sync_copy(rows_v, out_hbm.at[pl.ds(base, _BPW)])

    one_table(uidx_hbm, utab_hbm, up_hbm)
    one_table(iidx_hbm, itab_hbm, ip_hbm)


_gather = pl.kernel(
    _gather_body,
    out_type=[
        jax.ShapeDtypeStruct((BATCH, PAIRW), jnp.float32),
        jax.ShapeDtypeStruct((BATCH, PAIRW), jnp.float32),
    ],
    mesh=plsc.VectorSubcoreMesh(core_axis_name="c", subcore_axis_name="s"),
    scratch_types=[
        pltpu.VMEM((_NCHUNK, CHUNK), jnp.int32),
        pltpu.VMEM((_BPW, PAIRW), jnp.float32),
        pltpu.SemaphoreType.DMA,
    ],
    compiler_params=pltpu.CompilerParams(use_tc_tiling_on_sc=True),
)


def _mlp_body(up, ip, uh, ih, w1a, w1b, b1, w2, b2, w3, b3, out):
    uhc = uh[...][:, None]
    ihc = ih[...][:, None]
    ue = up[:, :EMBED] * (1.0 - uhc) + up[:, EMBED:] * uhc
    ie = ip[:, :EMBED] * (1.0 - ihc) + ip[:, EMBED:] * ihc
    h = jnp.dot(ue, w1a[...], preferred_element_type=jnp.float32)
    h = h + jnp.dot(ie, w1b[...], preferred_element_type=jnp.float32)
    h = jnp.maximum(h + b1[...], 0.0)
    h = jnp.maximum(
        jnp.dot(h, w2[...], preferred_element_type=jnp.float32) + b2[...], 0.0)
    o = jnp.dot(h, w3[...], preferred_element_type=jnp.float32)
    out[...] = o[:, 0] + b3[...][0, 0]


_BS = 2048

_mlp = pl.pallas_call(
    _mlp_body,
    grid=(BATCH // _BS,),
    in_specs=[
        pl.BlockSpec((_BS, PAIRW), lambda i: (i, 0)),
        pl.BlockSpec((_BS, PAIRW), lambda i: (i, 0)),
        pl.BlockSpec((_BS,), lambda i: (i,)),
        pl.BlockSpec((_BS,), lambda i: (i,)),
        pl.BlockSpec((EMBED, HID1), lambda i: (0, 0)),
        pl.BlockSpec((EMBED, HID1), lambda i: (0, 0)),
        pl.BlockSpec((1, HID1), lambda i: (0, 0)),
        pl.BlockSpec((HID1, HID2), lambda i: (0, 0)),
        pl.BlockSpec((1, HID2), lambda i: (0, 0)),
        pl.BlockSpec((HID2, 1), lambda i: (0, 0)),
        pl.BlockSpec((1, 1), lambda i: (0, 0)),
    ],
    out_specs=pl.BlockSpec((_BS,), lambda i: (i,)),
    out_shape=jax.ShapeDtypeStruct((BATCH,), jnp.float32),
    compiler_params=pltpu.CompilerParams(dimension_semantics=("arbitrary",)),
)


def kernel(users, items, user_table, item_table, W1, b1, W2, b2, W3, b3):
    users32 = users.astype(jnp.int32)
    items32 = items.astype(jnp.int32)
    uidx = (users32 >> 1).reshape(BATCH // CHUNK, CHUNK)
    iidx = (items32 >> 1).reshape(BATCH // CHUNK, CHUNK)
    uh = (users32 & 1).astype(jnp.float32)
    ih = (items32 & 1).astype(jnp.float32)
    tpose = _make_tpose(user_table.shape[0])
    tu = tpose(user_table.T)
    ti = tpose(item_table.T)
    up, ip = _gather(uidx, iidx, tu, ti)
    return _mlp(up, ip, uh, ih, W1[:EMBED], W1[EMBED:], b1.reshape(1, HID1),
                W2, b2.reshape(1, HID2), W3, b3.reshape(1, 1))


# speedup vs baseline: 1.7939x; 1.7939x over previous
import jax
import jax.numpy as jnp
from jax import lax
from jax.experimental import pallas as pl
from jax.experimental.pallas import tpu as pltpu
from jax.experimental.pallas import tpu_sc as plsc

BATCH = 16384
EMBED = 64
PAIRW = 128
HID1 = 128
HID2 = 64
CHUNK = 128

_info = plsc.get_sparse_core_info()
_NC, _NS = _info.num_cores, _info.num_subcores
_NW = _NC * _NS
_BPW = BATCH // _NW
_NCHUNK = _BPW // CHUNK

_TCOLS = 16384


def _tpose_body(tt, out):
    t = lax.dot_general(tt[...], jnp.eye(EMBED, dtype=jnp.float32),
                        (((0,), (0,)), ((), ())),
                        preferred_element_type=jnp.float32)
    out[...] = jnp.concatenate([t[:_TCOLS // 2], t[_TCOLS // 2:]], axis=1)


def _make_tpose(nrows):
    nblk = pl.cdiv(nrows, _TCOLS)
    return pl.pallas_call(
        _tpose_body,
        grid=(nblk,),
        in_specs=[pl.BlockSpec((EMBED, _TCOLS), lambda i: (0, i))],
        out_specs=pl.BlockSpec((_TCOLS // 2, PAIRW), lambda i: (i, 0)),
        out_shape=jax.ShapeDtypeStruct(
            (nblk * (_TCOLS // 2), PAIRW), jnp.float32),
        compiler_params=pltpu.CompilerParams(
            dimension_semantics=("arbitrary",)),
    )


def _gather_body(uidx_hbm, iidx_hbm, utab_hbm, itab_hbm, up_hbm, ip_hbm,
                 idx_v, rows_v, sem):
    wid = lax.axis_index("s") * _NC + lax.axis_index("c")
    base = wid * _BPW
    row0 = wid * _NCHUNK

    def one_table(idx_hbm, tab_hbm, out_hbm):
        pltpu.sync_copy(idx_hbm.at[pl.ds(row0, _NCHUNK)], idx_v)
        copies = [
            pltpu.async_copy(tab_hbm.at[idx_v.at[j]],
                             rows_v.at[pl.ds(j * CHUNK, CHUNK)], sem)
            for j in range(_NCHUNK)
        ]
        for c in copies:
            c.wait()
        pltpu.sync_copy(rows_v, out_hbm.at[pl.ds(base, _BPW)])

    one_table(uidx_hbm, utab_hbm, up_hbm)
    one_table(iidx_hbm, itab_hbm, ip_hbm)


_gather = pl.kernel(
    _gather_body,
    out_type=[
        jax.ShapeDtypeStruct((BATCH, PAIRW), jnp.float32),
        jax.ShapeDtypeStruct((BATCH, PAIRW), jnp.float32),
    ],
    mesh=plsc.VectorSubcoreMesh(core_axis_name="c", subcore_axis_name="s"),
    scratch_types=[
        pltpu.VMEM((_NCHUNK, CHUNK), jnp.int32),
        pltpu.VMEM((_BPW, PAIRW), jnp.float32),
        pltpu.SemaphoreType.DMA,
    ],
    compiler_params=pltpu.CompilerParams(use_tc_tiling_on_sc=True),
)


def _mlp_body(up, ip, uh, ih, w1a, w1b, b1, w2, b2, w3, b3, out):
    uhc = uh[...][:, None]
    ihc = ih[...][:, None]
    ue = up[:, :EMBED] * (1.0 - uhc) + up[:, EMBED:] * uhc
    ie = ip[:, :EMBED] * (1.0 - ihc) + ip[:, EMBED:] * ihc
    h = jnp.dot(ue, w1a[...], preferred_element_type=jnp.float32)
    h = h + jnp.dot(ie, w1b[...], preferred_element_type=jnp.float32)
    h = jnp.maximum(h + b1[...], 0.0)
    h = jnp.maximum(
        jnp.dot(h, w2[...], preferred_element_type=jnp.float32) + b2[...], 0.0)
    o = jnp.dot(h, w3[...], preferred_element_type=jnp.float32)
    out[...] = o[:, 0] + b3[...][0, 0]


_BS = 2048

_mlp = pl.pallas_call(
    _mlp_body,
    grid=(BATCH // _BS,),
    in_specs=[
        pl.BlockSpec((_BS, PAIRW), lambda i: (i, 0)),
        pl.BlockSpec((_BS, PAIRW), lambda i: (i, 0)),
        pl.BlockSpec((_BS,), lambda i: (i,)),
        pl.BlockSpec((_BS,), lambda i: (i,)),
        pl.BlockSpec((EMBED, HID1), lambda i: (0, 0)),
        pl.BlockSpec((EMBED, HID1), lambda i: (0, 0)),
        pl.BlockSpec((1, HID1), lambda i: (0, 0)),
        pl.BlockSpec((HID1, HID2), lambda i: (0, 0)),
        pl.BlockSpec((1, HID2), lambda i: (0, 0)),
        pl.BlockSpec((HID2, 1), lambda i: (0, 0)),
        pl.BlockSpec((1, 1), lambda i: (0, 0)),
    ],
    out_specs=pl.BlockSpec((_BS,), lambda i: (i,)),
    out_shape=jax.ShapeDtypeStruct((BATCH,), jnp.float32),
    compiler_params=pltpu.CompilerParams(dimension_semantics=("arbitrary",)),
)


def kernel(users, items, user_table, item_table, W1, b1, W2, b2, W3, b3):
    users32 = users.astype(jnp.int32)
    items32 = items.astype(jnp.int32)
    half = _TCOLS // 2
    uslot = (users32 // _TCOLS) * half + (users32 & (half - 1))
    islot = (items32 // _TCOLS) * half + (items32 & (half - 1))
    uidx = uslot.reshape(BATCH // CHUNK, CHUNK)
    iidx = islot.reshape(BATCH // CHUNK, CHUNK)
    uh = ((users32 // half) & 1).astype(jnp.float32)
    ih = ((items32 // half) & 1).astype(jnp.float32)
    tpose = _make_tpose(user_table.shape[0])
    tu = tpose(user_table.T)
    ti = tpose(item_table.T)
    up, ip = _gather(uidx, iidx, tu, ti)
    return _mlp(up, ip, uh, ih, W1[:EMBED], W1[EMBED:], b1.reshape(1, HID1),
                W2, b2.reshape(1, HID2), W3, b3.reshape(1, 1))


# speedup vs baseline: 1.8960x; 1.0569x over previous
import jax
import jax.numpy as jnp
from jax import lax
from jax.experimental import pallas as pl
from jax.experimental.pallas import tpu as pltpu
from jax.experimental.pallas import tpu_sc as plsc

BATCH = 16384
EMBED = 64
PAIRW = 128
HID1 = 128
HID2 = 64
CHUNK = 128

_info = plsc.get_sparse_core_info()
_NC, _NS = _info.num_cores, _info.num_subcores
_NW = _NC * _NS
_BPW = BATCH // _NW
_NCHUNK = _BPW // CHUNK

_TCOLS = 32768


def _tpose_body(tt, out):
    t = lax.dot_general(tt[...], jnp.eye(EMBED, dtype=jnp.float32),
                        (((0,), (0,)), ((), ())),
                        preferred_element_type=jnp.float32)
    out[...] = jnp.concatenate([t[:_TCOLS // 2], t[_TCOLS // 2:]], axis=1)


def _make_tpose(nrows):
    nblk = pl.cdiv(nrows, _TCOLS)
    return pl.pallas_call(
        _tpose_body,
        grid=(nblk,),
        in_specs=[pl.BlockSpec((EMBED, _TCOLS), lambda i: (0, i))],
        out_specs=pl.BlockSpec((_TCOLS // 2, PAIRW), lambda i: (i, 0)),
        out_shape=jax.ShapeDtypeStruct(
            (nblk * (_TCOLS // 2), PAIRW), jnp.float32),
        compiler_params=pltpu.CompilerParams(
            dimension_semantics=("arbitrary",)),
    )


def _gather_body(uidx_hbm, iidx_hbm, utab_hbm, itab_hbm, up_hbm, ip_hbm,
                 idx_v, rows_v, sem):
    wid = lax.axis_index("s") * _NC + lax.axis_index("c")
    base = wid * _BPW
    row0 = wid * _NCHUNK

    def one_table(idx_hbm, tab_hbm, out_hbm):
        pltpu.sync_copy(idx_hbm.at[pl.ds(row0, _NCHUNK)], idx_v)
        copies = [
            pltpu.async_copy(tab_hbm.at[idx_v.at[j]],
                             rows_v.at[pl.ds(j * CHUNK, CHUNK)], sem)
            for j in range(_NCHUNK)
        ]
        for c in copies:
            c.wait()
        pltpu.sync_copy(rows_v, out_hbm.at[pl.ds(base, _BPW)])

    one_table(uidx_hbm, utab_hbm, up_hbm)
    one_table(iidx_hbm, itab_hbm, ip_hbm)


_gather = pl.kernel(
    _gather_body,
    out_type=[
        jax.ShapeDtypeStruct((BATCH, PAIRW), jnp.float32),
        jax.ShapeDtypeStruct((BATCH, PAIRW), jnp.float32),
    ],
    mesh=plsc.VectorSubcoreMesh(core_axis_name="c", subcore_axis_name="s"),
    scratch_types=[
        pltpu.VMEM((_NCHUNK, CHUNK), jnp.int32),
        pltpu.VMEM((_BPW, PAIRW), jnp.float32),
        pltpu.SemaphoreType.DMA,
    ],
    compiler_params=pltpu.CompilerParams(use_tc_tiling_on_sc=True),
)


def _mlp_body(up, ip, uh, ih, w1a, w1b, b1, w2, b2, w3, b3, out):
    uhc = uh[...][:, None]
    ihc = ih[...][:, None]
    ue = up[:, :EMBED] * (1.0 - uhc) + up[:, EMBED:] * uhc
    ie = ip[:, :EMBED] * (1.0 - ihc) + ip[:, EMBED:] * ihc
    h = jnp.dot(ue, w1a[...], preferred_element_type=jnp.float32)
    h = h + jnp.dot(ie, w1b[...], preferred_element_type=jnp.float32)
    h = jnp.maximum(h + b1[...], 0.0)
    h = jnp.maximum(
        jnp.dot(h, w2[...], preferred_element_type=jnp.float32) + b2[...], 0.0)
    o = jnp.dot(h, w3[...], preferred_element_type=jnp.float32)
    out[...] = o[:, 0] + b3[...][0, 0]


_BS = 2048

_mlp = pl.pallas_call(
    _mlp_body,
    grid=(BATCH // _BS,),
    in_specs=[
        pl.BlockSpec((_BS, PAIRW), lambda i: (i, 0)),
        pl.BlockSpec((_BS, PAIRW), lambda i: (i, 0)),
        pl.BlockSpec((_BS,), lambda i: (i,)),
        pl.BlockSpec((_BS,), lambda i: (i,)),
        pl.BlockSpec((EMBED, HID1), lambda i: (0, 0)),
        pl.BlockSpec((EMBED, HID1), lambda i: (0, 0)),
        pl.BlockSpec((1, HID1), lambda i: (0, 0)),
        pl.BlockSpec((HID1, HID2), lambda i: (0, 0)),
        pl.BlockSpec((1, HID2), lambda i: (0, 0)),
        pl.BlockSpec((HID2, 1), lambda i: (0, 0)),
        pl.BlockSpec((1, 1), lambda i: (0, 0)),
    ],
    out_specs=pl.BlockSpec((_BS,), lambda i: (i,)),
    out_shape=jax.ShapeDtypeStruct((BATCH,), jnp.float32),
    compiler_params=pltpu.CompilerParams(dimension_semantics=("arbitrary",)),
)


def kernel(users, items, user_table, item_table, W1, b1, W2, b2, W3, b3):
    users32 = users.astype(jnp.int32)
    items32 = items.astype(jnp.int32)
    half = _TCOLS // 2
    uslot = (users32 // _TCOLS) * half + (users32 & (half - 1))
    islot = (items32 // _TCOLS) * half + (items32 & (half - 1))
    uidx = uslot.reshape(BATCH // CHUNK, CHUNK)
    iidx = islot.reshape(BATCH // CHUNK, CHUNK)
    uh = ((users32 // half) & 1).astype(jnp.float32)
    ih = ((items32 // half) & 1).astype(jnp.float32)
    tpose = _make_tpose(user_table.shape[0])
    tu = tpose(user_table.T)
    ti = tpose(item_table.T)
    up, ip = _gather(uidx, iidx, tu, ti)
    return _mlp(up, ip, uh, ih, W1[:EMBED], W1[EMBED:], b1.reshape(1, HID1),
                W2, b2.reshape(1, HID2), W3, b3.reshape(1, 1))


# speedup vs baseline: 1.9193x; 1.0123x over previous
import jax
import jax.numpy as jnp
from jax import lax
from jax.experimental import pallas as pl
from jax.experimental.pallas import tpu as pltpu
from jax.experimental.pallas import tpu_sc as plsc

BATCH = 16384
EMBED = 64
PAIRW = 128
HID1 = 128
HID2 = 64
CHUNK = 128

_info = plsc.get_sparse_core_info()
_NC, _NS = _info.num_cores, _info.num_subcores
_NW = _NC * _NS
_BPW = BATCH // _NW
_NCHUNK = _BPW // CHUNK

_TCOLS = 32768


def _tpose_body(tt, out):
    t = lax.dot_general(tt[...], jnp.eye(EMBED, dtype=jnp.float32),
                        (((0,), (0,)), ((), ())),
                        preferred_element_type=jnp.float32)
    out[...] = jnp.concatenate([t[:_TCOLS // 2], t[_TCOLS // 2:]], axis=1)


def _make_tpose(nrows):
    nblk = pl.cdiv(nrows, _TCOLS)
    return pl.pallas_call(
        _tpose_body,
        grid=(nblk,),
        in_specs=[pl.BlockSpec((EMBED, _TCOLS), lambda i: (0, i))],
        out_specs=pl.BlockSpec((_TCOLS // 2, PAIRW), lambda i: (i, 0)),
        out_shape=jax.ShapeDtypeStruct(
            (nblk * (_TCOLS // 2), PAIRW), jnp.float32),
        compiler_params=pltpu.CompilerParams(
            dimension_semantics=("arbitrary",)),
    )


def _gather_body(idx_hbm, tab_hbm, out_hbm, idx_v, rows_v, sem):
    wid = lax.axis_index("s") * _NC + lax.axis_index("c")
    base = wid * _BPW
    row0 = wid * _NCHUNK
    pltpu.sync_copy(idx_hbm.at[pl.ds(row0, _NCHUNK)], idx_v)
    copies = [
        pltpu.async_copy(tab_hbm.at[idx_v.at[j]],
                         rows_v.at[pl.ds(j * CHUNK, CHUNK)], sem)
        for j in range(_NCHUNK)
    ]
    for c in copies:
        c.wait()
    pltpu.sync_copy(rows_v, out_hbm.at[pl.ds(base, _BPW)])


_gather = pl.kernel(
    _gather_body,
    out_type=jax.ShapeDtypeStruct((BATCH, PAIRW), jnp.float32),
    mesh=plsc.VectorSubcoreMesh(core_axis_name="c", subcore_axis_name="s"),
    scratch_types=[
        pltpu.VMEM((_NCHUNK, CHUNK), jnp.int32),
        pltpu.VMEM((_BPW, PAIRW), jnp.float32),
        pltpu.SemaphoreType.DMA,
    ],
    compiler_params=pltpu.CompilerParams(use_tc_tiling_on_sc=True),
)


def _mlp_body(up, ip, uh, ih, w1a, w1b, b1, w2, b2, w3, b3, out):
    uhc = uh[...][:, None]
    ihc = ih[...][:, None]
    ue = up[:, :EMBED] * (1.0 - uhc) + up[:, EMBED:] * uhc
    ie = ip[:, :EMBED] * (1.0 - ihc) + ip[:, EMBED:] * ihc
    h = jnp.dot(ue, w1a[...], preferred_element_type=jnp.float32)
    h = h + jnp.dot(ie, w1b[...], preferred_element_type=jnp.float32)
    h = jnp.maximum(h + b1[...], 0.0)
    h = jnp.maximum(
        jnp.dot(h, w2[...], preferred_element_type=jnp.float32) + b2[...], 0.0)
    o = jnp.dot(h, w3[...], preferred_element_type=jnp.float32)
    out[...] = o[:, 0] + b3[...][0, 0]


_BS = 2048

_mlp = pl.pallas_call(
    _mlp_body,
    grid=(BATCH // _BS,),
    in_specs=[
        pl.BlockSpec((_BS, PAIRW), lambda i: (i, 0)),
        pl.BlockSpec((_BS, PAIRW), lambda i: (i, 0)),
        pl.BlockSpec((_BS,), lambda i: (i,)),
        pl.BlockSpec((_BS,), lambda i: (i,)),
        pl.BlockSpec((EMBED, HID1), lambda i: (0, 0)),
        pl.BlockSpec((EMBED, HID1), lambda i: (0, 0)),
        pl.BlockSpec((1, HID1), lambda i: (0, 0)),
        pl.BlockSpec((HID1, HID2), lambda i: (0, 0)),
        pl.BlockSpec((1, HID2), lambda i: (0, 0)),
        pl.BlockSpec((HID2, 1), lambda i: (0, 0)),
        pl.BlockSpec((1, 1), lambda i: (0, 0)),
    ],
    out_specs=pl.BlockSpec((_BS,), lambda i: (i,)),
    out_shape=jax.ShapeDtypeStruct((BATCH,), jnp.float32),
    compiler_params=pltpu.CompilerParams(dimension_semantics=("arbitrary",)),
)


def kernel(users, items, user_table, item_table, W1, b1, W2, b2, W3, b3):
    users32 = users.astype(jnp.int32)
    items32 = items.astype(jnp.int32)
    half = _TCOLS // 2
    uslot = (users32 // _TCOLS) * half + (users32 & (half - 1))
    islot = (items32 // _TCOLS) * half + (items32 & (half - 1))
    uidx = uslot.reshape(BATCH // CHUNK, CHUNK)
    iidx = islot.reshape(BATCH // CHUNK, CHUNK)
    uh = ((users32 // half) & 1).astype(jnp.float32)
    ih = ((items32 // half) & 1).astype(jnp.float32)
    tpose = _make_tpose(user_table.shape[0])
    tu = tpose(user_table.T)
    up = _gather(uidx, tu)
    ti = tpose(item_table.T)
    ip = _gather(iidx, ti)
    return _mlp(up, ip, uh, ih, W1[:EMBED], W1[EMBED:], b1.reshape(1, HID1),
                W2, b2.reshape(1, HID2), W3, b3.reshape(1, 1))
